# Initial kernel scaffold; baseline (speedup 1.0000x reference)
import jax
import jax.numpy as jnp
from jax.experimental import pallas as pl


def kernel(x, emb):
    raise NotImplementedError("write your pallas kernel here")


# speedup vs baseline: 1.7232x; 1.7232x over previous
import jax
import jax.numpy as jnp
from jax.experimental import pallas as pl

BLK_S = 512


def _add_kernel(x_ref, e_ref, o_ref):
    o_ref[...] = x_ref[...] + e_ref[...][None, :, :]


def kernel(x, emb):
    b, s, d = x.shape
    grid = (s // BLK_S,)
    return pl.pallas_call(
        _add_kernel,
        grid=grid,
        in_specs=[
            pl.BlockSpec((b, BLK_S, d), lambda i: (0, i, 0)),
            pl.BlockSpec((BLK_S, d), lambda i: (i, 0)),
        ],
        out_specs=pl.BlockSpec((b, BLK_S, d), lambda i: (0, i, 0)),
        out_shape=jax.ShapeDtypeStruct((b, s, d), x.dtype),
    )(x, emb)
